# baseline (speedup 1.0000x reference)
import jax
import jax.numpy as jnp
from jax.experimental import pallas as pl
from jax.experimental.pallas import tpu as pltpu

_TOP_K = 8
_N_EXPERTS = 64
_DIM = 768


def _topk_from_logits(logits):
    vals = logits.T
    iota = jax.lax.broadcasted_iota(jnp.int32, vals.shape, 0)
    m = None
    top_vals = []
    top_idx = []
    for _ in range(_TOP_K):
        mk = jnp.max(vals, axis=0, keepdims=True)
        ik = jnp.min(
            jnp.where(vals == mk, iota, _N_EXPERTS), axis=0, keepdims=True
        )
        if m is None:
            m = mk
        top_vals.append(jnp.exp(mk - m))
        top_idx.append(ik)
        vals = jnp.where(iota == ik, -jnp.inf, vals)
    w = jnp.concatenate(top_vals, axis=0)
    i = jnp.concatenate(top_idx, axis=0)
    w = w / jnp.sum(w, axis=0, keepdims=True)
    return i.T, w.T


_SUB = 512


_SIZES = (512, 512, 1024, 2048) + (4096,) * 7
_OFFS = tuple(sum(_SIZES[:i]) for i in range(len(_SIZES)))
_BUFTOK = 4096


def _chunk_body(logits, idx_ref, w_ref, off, sz):
    for s in range(sz // _SUB):
        i, w = _topk_from_logits(logits[s * _SUB : (s + 1) * _SUB])
        idx_ref[off + s * _SUB : off + (s + 1) * _SUB, :] = i
        w_ref[off + s * _SUB : off + (s + 1) * _SUB, :] = w


def _gate_kernel(x_hbm, wt_ref, idx_ref, w_ref, buf, sem):
    wt = wt_ref[...]
    n = len(_SIZES)

    def copy(i):
        off, sz = _OFFS[i], _SIZES[i]
        return pltpu.make_async_copy(
            x_hbm.at[pl.ds(off, sz), :],
            buf.at[i % 2, pl.ds(0, sz), :],
            sem.at[i % 2],
        )

    copy(0).start()
    copy(1).start()
    for i in range(n):
        copy(i).wait()
        off, sz = _OFFS[i], _SIZES[i]
        logits = jnp.dot(
            buf[i % 2, :sz, :], wt, preferred_element_type=jnp.float32
        )
        _chunk_body(logits, idx_ref, w_ref, off, sz)
        if i + 2 < n:
            copy(i + 2).start()


@jax.jit
def _gate(x, wt):
    n_tokens = x.shape[0]
    idx, w = pl.pallas_call(
        _gate_kernel,
        in_specs=[
            pl.BlockSpec(memory_space=pl.ANY),
            pl.BlockSpec(memory_space=pltpu.VMEM),
        ],
        out_specs=[
            pl.BlockSpec(memory_space=pltpu.VMEM),
            pl.BlockSpec(memory_space=pltpu.VMEM),
        ],
        out_shape=[
            jax.ShapeDtypeStruct((n_tokens, _TOP_K), jnp.int32),
            jax.ShapeDtypeStruct((n_tokens, _TOP_K), jnp.float32),
        ],
        scratch_shapes=[
            pltpu.VMEM((2, _BUFTOK, _DIM), jnp.float32),
            pltpu.SemaphoreType.DMA((2,)),
        ],
    )(x, wt)
    return idx, w


def kernel(hidden_states, weight):
    bsz, seq_len, h = hidden_states.shape
    x = hidden_states.reshape(-1, h)
    idx, w = _gate(x, weight.T)
    return idx, w, jnp.float32(0.0)


# speedup vs baseline: 1.1455x; 1.1455x over previous
import jax
import jax.numpy as jnp
from jax.experimental import pallas as pl
from jax.experimental.pallas import tpu as pltpu

_TOP_K = 8
_N_EXPERTS = 64
_DIM = 768


def _topk_from_logits(logits):
    vals = logits.T
    iota = jax.lax.broadcasted_iota(jnp.int32, vals.shape, 0)
    m = None
    top_vals = []
    top_idx = []
    for k in range(_TOP_K):
        mk = jnp.max(vals, axis=0, keepdims=True)
        ik = jnp.min(
            jnp.where(vals == mk, iota, _N_EXPERTS), axis=0, keepdims=True
        )
        if m is None:
            m = mk
            top_vals.append(jnp.ones_like(mk))
        else:
            top_vals.append(jnp.exp(mk - m))
        top_idx.append(ik)
        if k + 1 < _TOP_K:
            vals = jnp.where(iota == ik, -jnp.inf, vals)
    w = jnp.concatenate(top_vals, axis=0)
    i = jnp.concatenate(top_idx, axis=0)
    w = w / jnp.sum(w, axis=0, keepdims=True)
    return i.T, w.T


_SUB = 512


def _gate_kernel(x_ref, wt_ref, idx_ref, w_ref):
    logits = jnp.dot(
        x_ref[...], wt_ref[...], preferred_element_type=jnp.float32
    )
    tb = logits.shape[0]
    for s in range(tb // _SUB):
        i, w = _topk_from_logits(logits[s * _SUB : (s + 1) * _SUB])
        idx_ref[s * _SUB : (s + 1) * _SUB, :] = i
        w_ref[s * _SUB : (s + 1) * _SUB, :] = w


@jax.jit
def _gate(x, wt):
    n_tokens = x.shape[0]
    tb = 4096
    grid = (n_tokens // tb,)
    idx, w = pl.pallas_call(
        _gate_kernel,
        grid=grid,
        in_specs=[
            pl.BlockSpec((tb, _DIM), lambda i: (i, 0)),
            pl.BlockSpec((_DIM, _N_EXPERTS), lambda i: (0, 0)),
        ],
        out_specs=[
            pl.BlockSpec((tb, _TOP_K), lambda i: (i, 0)),
            pl.BlockSpec((tb, _TOP_K), lambda i: (i, 0)),
        ],
        out_shape=[
            jax.ShapeDtypeStruct((n_tokens, _TOP_K), jnp.int32),
            jax.ShapeDtypeStruct((n_tokens, _TOP_K), jnp.float32),
        ],
        compiler_params=pltpu.CompilerParams(
            dimension_semantics=("arbitrary",),
        ),
    )(x, wt)
    return idx, w


def kernel(hidden_states, weight):
    bsz, seq_len, h = hidden_states.shape
    x = hidden_states.reshape(-1, h)
    idx, w = _gate(x, weight.T)
    return idx, w, jnp.float32(0.0)
